# Initial kernel scaffold; baseline (speedup 1.0000x reference)
import functools

import jax
import jax.numpy as jnp
from jax import lax
from jax.experimental import pallas as pl
from jax.experimental.pallas import tpu as pltpu
from jax.experimental.pallas import tpu_sc as plsc

_NC = 2
_NS = 16
_NW = _NC * _NS
_L = 16


def _degree_kernel(n, epw):
    nv = epw // _L
    mesh = plsc.VectorSubcoreMesh(core_axis_name="c", subcore_axis_name="s")

    @functools.partial(
        pl.kernel,
        mesh=mesh,
        out_type=jax.ShapeDtypeStruct((_NW, n), jnp.float32),
        scratch_types=[
            pltpu.VMEM((epw,), jnp.int32),
            pltpu.VMEM((n,), jnp.float32),
        ],
    )
    def deg_k(col_hbm, out_hbm, col_v, hist_v):
        wid = lax.axis_index("s") * _NC + lax.axis_index("c")
        base = wid * epw
        pltpu.sync_copy(col_hbm.at[pl.ds(base, epw)], col_v)

        zeros = jnp.zeros((_L,), jnp.float32)

        def _zero(i, carry):
            hist_v[pl.ds(i * _L, _L)] = zeros
            return carry

        lax.fori_loop(0, n // _L, _zero, None)

        ones = jnp.ones((_L,), jnp.float32)

        def _acc(i, carry):
            idx = col_v[pl.ds(i * _L, _L)]
            plsc.addupdate_scatter(hist_v, [idx], ones)
            return carry

        lax.fori_loop(0, nv, _acc, None)
        pltpu.sync_copy(hist_v, out_hbm.at[wid])

    return deg_k


def _tables_kernel(n):
    def body(h_ref, x_ref, w_ref, b_ref, out_ref):
        deg = jnp.sum(h_ref[...], axis=0, keepdims=True)
        dis = lax.rsqrt(deg)
        pq = lax.dot_general(
            w_ref[...], x_ref[...],
            (((1,), (1,)), ((), ())),
            preferred_element_type=jnp.float32,
        )
        pq = jnp.maximum(pq + b_ref[...], 0.0)
        p = pq[0:1]
        q = pq[1:2]
        out_ref[...] = jnp.concatenate([dis, dis * p, q], axis=0)

    return pl.pallas_call(
        body,
        out_shape=jax.ShapeDtypeStruct((3, n), jnp.float32),
    )


def _edge_kernel(e, n, epw):
    nv = epw // _L
    mesh = plsc.VectorSubcoreMesh(core_axis_name="c", subcore_axis_name="s")

    @functools.partial(
        pl.kernel,
        mesh=mesh,
        out_type=jax.ShapeDtypeStruct((e,), jnp.float32),
        scratch_types=[
            pltpu.VMEM((epw,), jnp.int32),
            pltpu.VMEM((epw,), jnp.int32),
            pltpu.VMEM((epw,), jnp.float32),
            pltpu.VMEM((epw,), jnp.float32),
            pltpu.VMEM((n,), jnp.float32),
            pltpu.VMEM((n,), jnp.float32),
            pltpu.VMEM((n,), jnp.float32),
        ],
    )
    def edge_k(row_hbm, col_hbm, ea_hbm, tab_hbm, out_hbm,
               row_v, col_v, ea_v, out_v, ta_v, tb_v, tc_v):
        wid = lax.axis_index("s") * _NC + lax.axis_index("c")
        base = wid * epw
        pltpu.sync_copy(tab_hbm.at[0], ta_v)
        pltpu.sync_copy(tab_hbm.at[1], tb_v)
        pltpu.sync_copy(tab_hbm.at[2], tc_v)
        pltpu.sync_copy(row_hbm.at[pl.ds(base, epw)], row_v)
        pltpu.sync_copy(col_hbm.at[pl.ds(base, epw)], col_v)
        pltpu.sync_copy(ea_hbm.at[pl.ds(base, epw)], ea_v)

        def _step(i, carry):
            sl = pl.ds(i * _L, _L)
            ir = row_v[sl]
            ic = col_v[sl]
            av = plsc.load_gather(ta_v, [ir])
            bv = plsc.load_gather(tb_v, [ic])
            cv = plsc.load_gather(tc_v, [ir])
            ev = ea_v[sl]
            out_v[sl] = ev * (av * bv) + ev * cv
            return carry

        lax.fori_loop(0, nv, _step, None)
        pltpu.sync_copy(out_v, out_hbm.at[pl.ds(base, epw)])

    return edge_k


def kernel(x, edge_index, edge_attr, p_w, p_b, q_w, q_b):
    n, _ = x.shape
    e = edge_attr.shape[0]
    epw = e // _NW
    assert e == epw * _NW and epw % _L == 0 and n % _L == 0

    row = edge_index[0].astype(jnp.int32)
    col = edge_index[1].astype(jnp.int32)
    pqw = jnp.concatenate([p_w, q_w], axis=0)
    pqb = jnp.concatenate([p_b, q_b], axis=0).reshape(2, 1)

    hists = _degree_kernel(n, epw)(col)
    tables = _tables_kernel(n)(hists, x, pqw, pqb)
    out = _edge_kernel(e, n, epw)(row, col, edge_attr, tables)
    return (edge_index, out)


# speedup vs baseline: 145.4753x; 145.4753x over previous
import functools

import jax
import jax.numpy as jnp
from jax import lax
from jax.experimental import pallas as pl
from jax.experimental.pallas import tpu as pltpu
from jax.experimental.pallas import tpu_sc as plsc

_NC = 2
_NS = 16
_NW = _NC * _NS
_L = 16


def _pad_lanes(n):
    return (n + 127) // 128 * 128


def _degree_kernel(n, epw):
    nv = epw // _L
    npad = _pad_lanes(n)
    mesh = plsc.VectorSubcoreMesh(core_axis_name="c", subcore_axis_name="s")

    @functools.partial(
        pl.kernel,
        mesh=mesh,
        out_type=jax.ShapeDtypeStruct((_NW * npad,), jnp.float32),
        scratch_types=[
            pltpu.VMEM((epw,), jnp.int32),
            pltpu.VMEM((npad,), jnp.float32),
        ],
        compiler_params=pltpu.CompilerParams(needs_layout_passes=False),
    )
    def deg_k(col_hbm, out_hbm, col_v, hist_v):
        wid = lax.axis_index("s") * _NC + lax.axis_index("c")
        pltpu.sync_copy(col_hbm.at[pl.ds(wid * epw, epw)], col_v)

        zeros = jnp.zeros((_L,), jnp.float32)

        def _zero(i, carry):
            hist_v[pl.ds(i * _L, _L)] = zeros
            return carry

        lax.fori_loop(0, npad // _L, _zero, None)

        ones = jnp.ones((_L,), jnp.float32)

        def _acc(i, carry):
            idx = col_v[pl.ds(i * _L, _L)]
            plsc.addupdate_scatter(hist_v, [idx], ones)
            return carry

        lax.fori_loop(0, nv, _acc, None)
        pltpu.sync_copy(hist_v, out_hbm.at[pl.ds(wid * npad, npad)])

    return deg_k


def _tables_kernel(n):
    npad = _pad_lanes(n)

    def body(h_ref, x_ref, w_ref, b_ref, ta_ref, tb_ref, tc_ref):
        deg = h_ref[pl.ds(0, npad)]
        for k in range(1, _NW):
            deg = deg + h_ref[pl.ds(k * npad, npad)]
        dis = lax.rsqrt(deg[:n])
        pq = lax.dot_general(
            w_ref[...], x_ref[...],
            (((1,), (1,)), ((), ())),
            preferred_element_type=jnp.float32,
        )
        pq = jnp.maximum(pq + b_ref[...], 0.0)
        p = pq[0]
        q = pq[1]
        ta_ref[...] = dis
        tb_ref[...] = dis * p
        tc_ref[...] = q

    return pl.pallas_call(
        body,
        out_shape=[
            jax.ShapeDtypeStruct((n,), jnp.float32),
            jax.ShapeDtypeStruct((n,), jnp.float32),
            jax.ShapeDtypeStruct((n,), jnp.float32),
        ],
    )


def _edge_kernel(e, n, epw):
    nv = epw // _L
    mesh = plsc.VectorSubcoreMesh(core_axis_name="c", subcore_axis_name="s")

    @functools.partial(
        pl.kernel,
        mesh=mesh,
        out_type=jax.ShapeDtypeStruct((e,), jnp.float32),
        scratch_types=[
            pltpu.VMEM((epw,), jnp.int32),
            pltpu.VMEM((epw,), jnp.int32),
            pltpu.VMEM((epw,), jnp.float32),
            pltpu.VMEM((epw,), jnp.float32),
            pltpu.VMEM((n,), jnp.float32),
            pltpu.VMEM((n,), jnp.float32),
            pltpu.VMEM((n,), jnp.float32),
        ],
        compiler_params=pltpu.CompilerParams(needs_layout_passes=False),
    )
    def edge_k(row_hbm, col_hbm, ea_hbm, ta_hbm, tb_hbm, tc_hbm, out_hbm,
               row_v, col_v, ea_v, out_v, ta_v, tb_v, tc_v):
        wid = lax.axis_index("s") * _NC + lax.axis_index("c")
        base = wid * epw
        pltpu.sync_copy(ta_hbm, ta_v)
        pltpu.sync_copy(tb_hbm, tb_v)
        pltpu.sync_copy(tc_hbm, tc_v)
        pltpu.sync_copy(row_hbm.at[pl.ds(base, epw)], row_v)
        pltpu.sync_copy(col_hbm.at[pl.ds(base, epw)], col_v)
        pltpu.sync_copy(ea_hbm.at[pl.ds(base, epw)], ea_v)

        def _step(i, carry):
            sl = pl.ds(i * _L, _L)
            ir = row_v[sl]
            ic = col_v[sl]
            av = plsc.load_gather(ta_v, [ir])
            bv = plsc.load_gather(tb_v, [ic])
            cv = plsc.load_gather(tc_v, [ir])
            ev = ea_v[sl]
            out_v[sl] = ev * (av * bv) + ev * cv
            return carry

        lax.fori_loop(0, nv, _step, None)
        pltpu.sync_copy(out_v, out_hbm.at[pl.ds(base, epw)])

    return edge_k


def kernel(x, edge_index, edge_attr, p_w, p_b, q_w, q_b):
    n, _ = x.shape
    e = edge_attr.shape[0]
    epw = e // _NW
    assert e == epw * _NW and epw % _L == 0 and epw % 8 == 0

    row = edge_index[0].astype(jnp.int32)
    col = edge_index[1].astype(jnp.int32)
    pqw = jnp.concatenate([p_w, q_w], axis=0)
    pqb = jnp.concatenate([p_b, q_b], axis=0).reshape(2, 1)

    hists = _degree_kernel(n, epw)(col)
    ta, tb, tc = _tables_kernel(n)(hists, x, pqw, pqb)
    out = _edge_kernel(e, n, epw)(row, col, edge_attr, ta, tb, tc)
    return (edge_index, out)
